# Initial kernel scaffold; baseline (speedup 1.0000x reference)
import jax
import jax.numpy as jnp
from jax.experimental import pallas as pl


def kernel(tensor):
    raise NotImplementedError("write your pallas kernel here")


# speedup vs baseline: 14.4956x; 14.4956x over previous
import functools

import numpy as np
import jax
import jax.numpy as jnp
from jax import lax
from jax.experimental import pallas as pl
from jax.experimental.pallas import tpu as pltpu
from jax.experimental.pallas import tpu_sc as plsc

NROW, NCOL = 128, 32768
NTOT = NROW * NCOL
_POS = np.float32(0.96) * np.float32(NTOT - 1)
K_RANK = int(np.floor(_POS))
FRAC = np.float32(_POS - np.floor(_POS))

NC, NS = 2, 16
NW = NC * NS
PER_W = NTOT // NW
CHUNK = 16384
NCHUNK = PER_W // CHUNK
VECS = CHUNK // 16
NB12 = 4096
NB3 = 256
INTMAX = np.int32(2**31 - 1)
SIGNBIT = np.int32(-2**31)


def _ikey16(x):
    u = lax.bitcast_convert_type(x, jnp.int32)
    s = lax.shift_right_logical(u, 31)
    return u ^ lax.shift_right_logical(0 - s, 1)


def _mesh():
    return plsc.VectorSubcoreMesh(core_axis_name="c", subcore_axis_name="s")


_SC_PARAMS = pltpu.CompilerParams(needs_layout_passes=False)


def _sc_pass1(flat):
    @functools.partial(
        pl.kernel, mesh=_mesh(), compiler_params=_SC_PARAMS,
        out_type=jax.ShapeDtypeStruct((NW, NB12 * 16), jnp.int32),
        scratch_types=[pltpu.VMEM((CHUNK,), jnp.float32),
                       pltpu.VMEM((NB12 * 16,), jnp.int32)])
    def k(x_hbm, h_hbm, buf, hist):
        wid = lax.axis_index("s") * NC + lax.axis_index("c")
        base = wid * PER_W
        lane = lax.iota(jnp.int32, 16)
        ones = jnp.ones((16,), jnp.int32)

        def zero(i, _):
            hist[pl.ds(i * 16, 16)] = jnp.zeros((16,), jnp.int32)
            return 0
        lax.fori_loop(0, NB12, zero, 0)

        def chunk(c, _):
            pltpu.sync_copy(x_hbm.at[pl.ds(base + c * CHUNK, CHUNK)], buf)

            def body(i, _):
                ikey = _ikey16(buf[pl.ds(i * 16, 16)])
                b = lax.shift_right_arithmetic(ikey, 20) + 2048
                plsc.addupdate_scatter(hist, [lax.shift_left(b, 4) | lane],
                                       ones)
                return 0
            lax.fori_loop(0, VECS, body, 0)
            return 0
        lax.fori_loop(0, NCHUNK, chunk, 0)
        pltpu.sync_copy(hist, h_hbm.at[wid])

    return k(flat)


def _sc_pass2(flat, sel1):
    @functools.partial(
        pl.kernel, mesh=_mesh(), compiler_params=_SC_PARAMS,
        out_type=(jax.ShapeDtypeStruct((NW, NB12 * 16), jnp.int32),
                  jax.ShapeDtypeStruct((NW, 16), jnp.int32)),
        scratch_types=[pltpu.VMEM((CHUNK,), jnp.float32),
                       pltpu.VMEM((NB12 * 16,), jnp.int32),
                       pltpu.VMEM((128,), jnp.int32),
                       pltpu.VMEM((16,), jnp.int32)])
    def k(x_hbm, sel_hbm, h_hbm, mn_hbm, buf, hist, selbuf, mnbuf):
        wid = lax.axis_index("s") * NC + lax.axis_index("c")
        base = wid * PER_W
        lane = lax.iota(jnp.int32, 16)
        ones = jnp.ones((16,), jnp.int32)
        pltpu.sync_copy(sel_hbm.at[0], selbuf)
        b1t = selbuf[pl.ds(0, 16)]

        def zero(i, _):
            hist[pl.ds(i * 16, 16)] = jnp.zeros((16,), jnp.int32)
            return 0
        lax.fori_loop(0, NB12, zero, 0)

        def chunk(c, minv):
            pltpu.sync_copy(x_hbm.at[pl.ds(base + c * CHUNK, CHUNK)], buf)

            def body(i, minv):
                ikey = _ikey16(buf[pl.ds(i * 16, 16)])
                b1 = lax.shift_right_arithmetic(ikey, 20) + 2048
                b2 = lax.shift_right_logical(ikey, 8) & 0xFFF
                plsc.addupdate_scatter(hist, [lax.shift_left(b2, 4) | lane],
                                       ones, mask=b1 == b1t)
                return jnp.minimum(minv, jnp.where(b1 > b1t, ikey, INTMAX))
            return lax.fori_loop(0, VECS, body, minv)

        minv = lax.fori_loop(0, NCHUNK, chunk,
                             jnp.full((16,), INTMAX, jnp.int32))
        mnbuf[...] = minv
        pltpu.sync_copy(hist, h_hbm.at[wid])
        pltpu.sync_copy(mnbuf, mn_hbm.at[wid])

    return k(flat, sel1)


def _sc_pass3(flat, sel2):
    @functools.partial(
        pl.kernel, mesh=_mesh(), compiler_params=_SC_PARAMS,
        out_type=(jax.ShapeDtypeStruct((NW, NB3 * 16), jnp.int32),
                  jax.ShapeDtypeStruct((NW, 16), jnp.int32)),
        scratch_types=[pltpu.VMEM((CHUNK,), jnp.float32),
                       pltpu.VMEM((NB3 * 16,), jnp.int32),
                       pltpu.VMEM((128,), jnp.int32),
                       pltpu.VMEM((128,), jnp.int32),
                       pltpu.VMEM((16,), jnp.int32)])
    def k(x_hbm, sel_hbm, h_hbm, mn_hbm, buf, hist, selb1, selb2, mnbuf):
        wid = lax.axis_index("s") * NC + lax.axis_index("c")
        base = wid * PER_W
        lane = lax.iota(jnp.int32, 16)
        ones = jnp.ones((16,), jnp.int32)
        pltpu.sync_copy(sel_hbm.at[0], selb1)
        pltpu.sync_copy(sel_hbm.at[1], selb2)
        b1t = selb1[pl.ds(0, 16)]
        b2t = selb2[pl.ds(0, 16)]

        def zero(i, _):
            hist[pl.ds(i * 16, 16)] = jnp.zeros((16,), jnp.int32)
            return 0
        lax.fori_loop(0, NB3, zero, 0)

        def chunk(c, minv):
            pltpu.sync_copy(x_hbm.at[pl.ds(base + c * CHUNK, CHUNK)], buf)

            def body(i, minv):
                ikey = _ikey16(buf[pl.ds(i * 16, 16)])
                b1 = lax.shift_right_arithmetic(ikey, 20) + 2048
                b2 = lax.shift_right_logical(ikey, 8) & 0xFFF
                b3 = ikey & 0xFF
                in1 = b1 == b1t
                plsc.addupdate_scatter(hist, [lax.shift_left(b3, 4) | lane],
                                       ones, mask=in1 & (b2 == b2t))
                return jnp.minimum(
                    minv, jnp.where(in1 & (b2 > b2t), ikey, INTMAX))
            return lax.fori_loop(0, VECS, body, minv)

        minv = lax.fori_loop(0, NCHUNK, chunk,
                             jnp.full((16,), INTMAX, jnp.int32))
        mnbuf[...] = minv
        pltpu.sync_copy(hist, h_hbm.at[wid])
        pltpu.sync_copy(mnbuf, mn_hbm.at[wid])

    return k(flat, sel2)


def _select_math(h, R, kt):
    f32 = jnp.float32
    rows = lax.broadcasted_iota(jnp.int32, (R, 128), 0)
    cols = lax.broadcasted_iota(jnp.int32, (R, 128), 1)
    bucket = (rows * 8 + lax.shift_right_logical(cols, 4)).astype(f32)
    gi = lax.shift_right_logical(
        lax.broadcasted_iota(jnp.int32, (128, 128), 0), 4)
    gj = lax.shift_right_logical(
        lax.broadcasted_iota(jnp.int32, (128, 128), 1), 4)
    same = (gi == gj).astype(f32)
    before = (gi < gj).astype(f32)
    hb = jnp.dot(h, same, preferred_element_type=f32,
                 precision=lax.Precision.HIGHEST)
    win = jnp.dot(h, before, preferred_element_type=f32,
                  precision=lax.Precision.HIGHEST)
    ri = lax.broadcasted_iota(jnp.int32, (R, R), 0)
    rj = lax.broadcasted_iota(jnp.int32, (R, R), 1)
    lower = (ri > rj).astype(f32)
    rs = jnp.broadcast_to(jnp.sum(h, axis=1, keepdims=True), (R, 128))
    rex = jnp.dot(lower, rs, preferred_element_type=f32,
                  precision=lax.Precision.HIGHEST)
    cb = rex + win
    cond = (cb <= kt) & (kt < cb + hb)
    neg = jnp.float32(-1.0)
    return (jnp.max(jnp.where(cond, bucket, neg)),
            jnp.max(jnp.where(cond, cb, neg)),
            jnp.max(jnp.where(cond, hb, neg)))


def _rows_to_out(vals):
    r = lax.broadcasted_iota(jnp.int32, (8, 128), 0)
    out = jnp.zeros((8, 128), jnp.float32)
    for i, v in enumerate(vals):
        out = out + jnp.where(r == i, v, 0.0)
    return out.astype(jnp.int32)


def _tc_select1(h1v):
    def body(h_ref, o_ref):
        h = jnp.sum(h_ref[...].astype(jnp.float32), axis=0)
        b, rex, cnt = _select_math(h, NB12 // 8, jnp.float32(K_RANK))
        o_ref[...] = _rows_to_out([b, rex, cnt])

    return pl.pallas_call(
        body, out_shape=jax.ShapeDtypeStruct((8, 128), jnp.int32))(h1v)


def _tc_select2(h2v, sel1):
    def body(h_ref, s_ref, o_ref):
        h = jnp.sum(h_ref[...].astype(jnp.float32), axis=0)
        b1 = s_ref[0, 0]
        r0 = s_ref[1, 0]
        kt = (K_RANK - r0).astype(jnp.float32)
        b2, rex, cnt = _select_math(h, NB12 // 8, kt)
        r01 = r0.astype(jnp.float32) + rex
        o_ref[...] = _rows_to_out([b1.astype(jnp.float32), b2, r01, cnt])

    return pl.pallas_call(
        body, out_shape=jax.ShapeDtypeStruct((8, 128), jnp.int32))(h2v, sel1)


def _tofloat(ik):
    bits = jnp.where(ik >= 0, ik, (~ik) | SIGNBIT)
    return lax.bitcast_convert_type(bits, jnp.float32)


def _tc_finalize(tensor, h3v, sel2, mina):
    grid = 16
    rows_blk = NROW // grid

    def body(x_ref, h_ref, s_ref, m_ref, o_ref):
        h = jnp.sum(h_ref[...].astype(jnp.float32), axis=0)
        b1 = s_ref[0, 0]
        b2 = s_ref[1, 0]
        r01 = s_ref[2, 0]
        cnt12 = s_ref[3, 0]
        jt = (K_RANK - r01).astype(jnp.float32)
        b3a, _, _ = _select_math(h, NB3 // 8, jt)
        b3b, _, _ = _select_math(h, NB3 // 8, jt + 1.0)
        prefix = (b1 - 2048) * 1048576 + b2 * 256
        ikey_k = prefix + b3a.astype(jnp.int32)
        ikey_k1_in = prefix + b3b.astype(jnp.int32)
        mmin = jnp.min(m_ref[...])
        have_b = (jt + 1.0) < cnt12.astype(jnp.float32)
        ikey_k1 = jnp.where(have_b, ikey_k1_in, mmin)
        vk = _tofloat(ikey_k)
        vk1 = _tofloat(ikey_k1)
        t = vk * (np.float32(1.0) - FRAC) + vk1 * FRAC
        clip = jnp.where(vk1 <= t, vk1, vk)
        o_ref[...] = jnp.minimum(x_ref[...], clip)

    return pl.pallas_call(
        body,
        grid=(grid,),
        in_specs=[
            pl.BlockSpec((rows_blk, NCOL), lambda i: (i, 0)),
            pl.BlockSpec((NW, NB3 * 16 // 128, 128), lambda i: (0, 0, 0)),
            pl.BlockSpec((8, 128), lambda i: (0, 0)),
            pl.BlockSpec((8, 128), lambda i: (0, 0)),
        ],
        out_specs=pl.BlockSpec((rows_blk, NCOL), lambda i: (i, 0)),
        out_shape=jax.ShapeDtypeStruct((NROW, NCOL), jnp.float32),
    )(tensor, h3v, sel2, mina)


def kernel(tensor):
    flat = tensor.reshape(-1)
    h1 = _sc_pass1(flat)
    sel1 = _tc_select1(h1.reshape(NW, NB12 * 16 // 128, 128))
    h2, mina2 = _sc_pass2(flat, sel1)
    sel2 = _tc_select2(h2.reshape(NW, NB12 * 16 // 128, 128), sel1)
    h3, mina3 = _sc_pass3(flat, sel2)
    mina = jnp.concatenate([mina2, mina3], axis=0).reshape(8, 128)
    return _tc_finalize(tensor, h3.reshape(NW, NB3 * 16 // 128, 128),
                        sel2, mina)


# speedup vs baseline: 17.4900x; 1.2066x over previous
import functools

import numpy as np
import jax
import jax.numpy as jnp
from jax import lax
from jax.experimental import pallas as pl
from jax.experimental.pallas import tpu as pltpu
from jax.experimental.pallas import tpu_sc as plsc

NROW, NCOL = 128, 32768
NTOT = NROW * NCOL
_POS = np.float32(0.96) * np.float32(NTOT - 1)
K_RANK = int(np.floor(_POS))
FRAC = np.float32(_POS - np.floor(_POS))

NC, NS = 2, 16
NW = NC * NS
PER_W = NTOT // NW
ROWS_PER_W = NROW // NW
RVECS = NCOL // 16
NB12 = 4096
NB3 = 256
INTMAX = np.int32(2**31 - 1)
SIGNBIT = np.int32(-2**31)


def _ikey16(x):
    u = lax.bitcast_convert_type(x, jnp.int32)
    s = lax.shift_right_logical(u, 31)
    return u ^ lax.shift_right_logical(0 - s, 1)


def _mesh():
    return plsc.VectorSubcoreMesh(core_axis_name="c", subcore_axis_name="s")


_SC_PARAMS = pltpu.CompilerParams(needs_layout_passes=False)


def _sc_pass1(flat):
    @functools.partial(
        pl.kernel, mesh=_mesh(), compiler_params=_SC_PARAMS,
        out_type=jax.ShapeDtypeStruct((NW, NB12 * 16), jnp.int32),
        scratch_types=[pltpu.VMEM((NCOL,), jnp.float32),
                       pltpu.VMEM((NB12 * 16,), jnp.int32)])
    def k(x_hbm, h_hbm, buf, hist):
        wid = lax.axis_index("s") * NC + lax.axis_index("c")
        lane = lax.iota(jnp.int32, 16)
        ones = jnp.ones((16,), jnp.int32)

        def zero(i, _):
            hist[pl.ds(i * 16, 16)] = jnp.zeros((16,), jnp.int32)
            return 0
        lax.fori_loop(0, NB12, zero, 0, unroll=8)

        for rr in range(ROWS_PER_W):
            pltpu.sync_copy(x_hbm.at[wid * ROWS_PER_W + rr], buf)

            def body(i, _):
                ikey = _ikey16(buf[pl.ds(i * 16, 16)])
                b = lax.shift_right_arithmetic(ikey, 20) + 2048
                plsc.addupdate_scatter(hist, [lax.shift_left(b, 4) | lane],
                                       ones)
                return 0
            lax.fori_loop(0, RVECS, body, 0, unroll=8)
        pltpu.sync_copy(hist, h_hbm.at[wid])

    return k(flat)


def _sc_pass2(flat, sel1):
    @functools.partial(
        pl.kernel, mesh=_mesh(), compiler_params=_SC_PARAMS,
        out_type=(jax.ShapeDtypeStruct((NW, NB12 * 16), jnp.int32),
                  jax.ShapeDtypeStruct((NW, 16), jnp.int32)),
        scratch_types=[pltpu.VMEM((NCOL,), jnp.float32),
                       pltpu.VMEM((NB12 * 16,), jnp.int32),
                       pltpu.VMEM((128,), jnp.int32),
                       pltpu.VMEM((16,), jnp.int32)])
    def k(x_hbm, sel_hbm, h_hbm, mn_hbm, buf, hist, selbuf, mnbuf):
        wid = lax.axis_index("s") * NC + lax.axis_index("c")
        lane = lax.iota(jnp.int32, 16)
        ones = jnp.ones((16,), jnp.int32)
        pltpu.sync_copy(sel_hbm.at[0], selbuf)
        b1t = selbuf[pl.ds(0, 16)]

        def zero(i, _):
            hist[pl.ds(i * 16, 16)] = jnp.zeros((16,), jnp.int32)
            return 0
        lax.fori_loop(0, NB12, zero, 0, unroll=8)

        minv = jnp.full((16,), INTMAX, jnp.int32)
        for rr in range(ROWS_PER_W):
            pltpu.sync_copy(x_hbm.at[wid * ROWS_PER_W + rr], buf)

            def body(i, minv):
                ikey = _ikey16(buf[pl.ds(i * 16, 16)])
                b1 = lax.shift_right_arithmetic(ikey, 20) + 2048
                b2 = lax.shift_right_logical(ikey, 8) & 0xFFF
                plsc.addupdate_scatter(hist, [lax.shift_left(b2, 4) | lane],
                                       ones, mask=b1 == b1t)
                return jnp.minimum(minv, jnp.where(b1 > b1t, ikey, INTMAX))
            minv = lax.fori_loop(0, RVECS, body, minv, unroll=8)
        mnbuf[...] = minv
        pltpu.sync_copy(hist, h_hbm.at[wid])
        pltpu.sync_copy(mnbuf, mn_hbm.at[wid])

    return k(flat, sel1)


def _sc_pass3(flat, sel2):
    @functools.partial(
        pl.kernel, mesh=_mesh(), compiler_params=_SC_PARAMS,
        out_type=(jax.ShapeDtypeStruct((NW, NB3 * 16), jnp.int32),
                  jax.ShapeDtypeStruct((NW, 16), jnp.int32)),
        scratch_types=[pltpu.VMEM((NCOL,), jnp.float32),
                       pltpu.VMEM((NB3 * 16,), jnp.int32),
                       pltpu.VMEM((128,), jnp.int32),
                       pltpu.VMEM((128,), jnp.int32),
                       pltpu.VMEM((16,), jnp.int32)])
    def k(x_hbm, sel_hbm, h_hbm, mn_hbm, buf, hist, selb1, selb2, mnbuf):
        wid = lax.axis_index("s") * NC + lax.axis_index("c")
        lane = lax.iota(jnp.int32, 16)
        ones = jnp.ones((16,), jnp.int32)
        pltpu.sync_copy(sel_hbm.at[0], selb1)
        pltpu.sync_copy(sel_hbm.at[1], selb2)
        b1t = selb1[pl.ds(0, 16)]
        b2t = selb2[pl.ds(0, 16)]

        def zero(i, _):
            hist[pl.ds(i * 16, 16)] = jnp.zeros((16,), jnp.int32)
            return 0
        lax.fori_loop(0, NB3, zero, 0, unroll=8)

        minv = jnp.full((16,), INTMAX, jnp.int32)
        for rr in range(ROWS_PER_W):
            pltpu.sync_copy(x_hbm.at[wid * ROWS_PER_W + rr], buf)

            def body(i, minv):
                ikey = _ikey16(buf[pl.ds(i * 16, 16)])
                b1 = lax.shift_right_arithmetic(ikey, 20) + 2048
                b2 = lax.shift_right_logical(ikey, 8) & 0xFFF
                b3 = ikey & 0xFF
                in1 = b1 == b1t
                plsc.addupdate_scatter(hist, [lax.shift_left(b3, 4) | lane],
                                       ones, mask=in1 & (b2 == b2t))
                return jnp.minimum(
                    minv, jnp.where(in1 & (b2 > b2t), ikey, INTMAX))
            minv = lax.fori_loop(0, RVECS, body, minv, unroll=8)
        mnbuf[...] = minv
        pltpu.sync_copy(hist, h_hbm.at[wid])
        pltpu.sync_copy(mnbuf, mn_hbm.at[wid])

    return k(flat, sel2)


def _select_math(h, R, kt):
    f32 = jnp.float32
    rows = lax.broadcasted_iota(jnp.int32, (R, 128), 0)
    cols = lax.broadcasted_iota(jnp.int32, (R, 128), 1)
    bucket = (rows * 8 + lax.shift_right_logical(cols, 4)).astype(f32)
    gi = lax.shift_right_logical(
        lax.broadcasted_iota(jnp.int32, (128, 128), 0), 4)
    gj = lax.shift_right_logical(
        lax.broadcasted_iota(jnp.int32, (128, 128), 1), 4)
    same = (gi == gj).astype(f32)
    before = (gi < gj).astype(f32)
    hb = jnp.dot(h, same, preferred_element_type=f32,
                 precision=lax.Precision.HIGHEST)
    win = jnp.dot(h, before, preferred_element_type=f32,
                  precision=lax.Precision.HIGHEST)
    ri = lax.broadcasted_iota(jnp.int32, (R, R), 0)
    rj = lax.broadcasted_iota(jnp.int32, (R, R), 1)
    lower = (ri > rj).astype(f32)
    rs = jnp.broadcast_to(jnp.sum(h, axis=1, keepdims=True), (R, 128))
    rex = jnp.dot(lower, rs, preferred_element_type=f32,
                  precision=lax.Precision.HIGHEST)
    cb = rex + win
    cond = (cb <= kt) & (kt < cb + hb)
    neg = jnp.float32(-1.0)
    return (jnp.max(jnp.where(cond, bucket, neg)),
            jnp.max(jnp.where(cond, cb, neg)),
            jnp.max(jnp.where(cond, hb, neg)))


def _rows_to_out(vals):
    r = lax.broadcasted_iota(jnp.int32, (8, 128), 0)
    out = jnp.zeros((8, 128), jnp.float32)
    for i, v in enumerate(vals):
        out = out + jnp.where(r == i, v, 0.0)
    return out.astype(jnp.int32)


def _tc_select1(h1v):
    def body(h_ref, o_ref):
        h = jnp.sum(h_ref[...].astype(jnp.float32), axis=0)
        b, rex, cnt = _select_math(h, NB12 // 8, jnp.float32(K_RANK))
        o_ref[...] = _rows_to_out([b, rex, cnt])

    return pl.pallas_call(
        body, out_shape=jax.ShapeDtypeStruct((8, 128), jnp.int32))(h1v)


def _tc_select2(h2v, sel1):
    def body(h_ref, s_ref, o_ref):
        h = jnp.sum(h_ref[...].astype(jnp.float32), axis=0)
        b1 = s_ref[0, 0]
        r0 = s_ref[1, 0]
        kt = (K_RANK - r0).astype(jnp.float32)
        b2, rex, cnt = _select_math(h, NB12 // 8, kt)
        r01 = r0.astype(jnp.float32) + rex
        o_ref[...] = _rows_to_out([b1.astype(jnp.float32), b2, r01, cnt])

    return pl.pallas_call(
        body, out_shape=jax.ShapeDtypeStruct((8, 128), jnp.int32))(h2v, sel1)


def _tofloat(ik):
    bits = jnp.where(ik >= 0, ik, (~ik) | SIGNBIT)
    return lax.bitcast_convert_type(bits, jnp.float32)


def _tc_finalize(tensor, h3v, sel2, mina):
    grid = 16
    rows_blk = NROW // grid

    def body(x_ref, h_ref, s_ref, m_ref, o_ref):
        h = jnp.sum(h_ref[...].astype(jnp.float32), axis=0)
        b1 = s_ref[0, 0]
        b2 = s_ref[1, 0]
        r01 = s_ref[2, 0]
        cnt12 = s_ref[3, 0]
        jt = (K_RANK - r01).astype(jnp.float32)
        b3a, _, _ = _select_math(h, NB3 // 8, jt)
        b3b, _, _ = _select_math(h, NB3 // 8, jt + 1.0)
        prefix = (b1 - 2048) * 1048576 + b2 * 256
        ikey_k = prefix + b3a.astype(jnp.int32)
        ikey_k1_in = prefix + b3b.astype(jnp.int32)
        mmin = jnp.min(m_ref[...])
        have_b = (jt + 1.0) < cnt12.astype(jnp.float32)
        ikey_k1 = jnp.where(have_b, ikey_k1_in, mmin)
        vk = _tofloat(ikey_k)
        vk1 = _tofloat(ikey_k1)
        t = vk * (np.float32(1.0) - FRAC) + vk1 * FRAC
        clip = jnp.where(vk1 <= t, vk1, vk)
        o_ref[...] = jnp.minimum(x_ref[...], clip)

    return pl.pallas_call(
        body,
        grid=(grid,),
        in_specs=[
            pl.BlockSpec((rows_blk, NCOL), lambda i: (i, 0)),
            pl.BlockSpec((NW, NB3 * 16 // 128, 128), lambda i: (0, 0, 0)),
            pl.BlockSpec((8, 128), lambda i: (0, 0)),
            pl.BlockSpec((8, 128), lambda i: (0, 0)),
        ],
        out_specs=pl.BlockSpec((rows_blk, NCOL), lambda i: (i, 0)),
        out_shape=jax.ShapeDtypeStruct((NROW, NCOL), jnp.float32),
    )(tensor, h3v, sel2, mina)


def kernel(tensor):
    h1 = _sc_pass1(tensor)
    sel1 = _tc_select1(h1.reshape(NW, NB12 * 16 // 128, 128))
    h2, mina2 = _sc_pass2(tensor, sel1)
    sel2 = _tc_select2(h2.reshape(NW, NB12 * 16 // 128, 128), sel1)
    h3, mina3 = _sc_pass3(tensor, sel2)
    mina = jnp.concatenate([mina2, mina3], axis=0).reshape(8, 128)
    return _tc_finalize(tensor, h3.reshape(NW, NB3 * 16 // 128, 128),
                        sel2, mina)


# speedup vs baseline: 35.2821x; 2.0173x over previous
import functools

import numpy as np
import jax
import jax.numpy as jnp
from jax import lax
from jax.experimental import pallas as pl
from jax.experimental.pallas import tpu as pltpu
from jax.experimental.pallas import tpu_sc as plsc

NROW, NCOL = 128, 32768
NTOT = NROW * NCOL
_POS = np.float32(0.96) * np.float32(NTOT - 1)
K_RANK = int(np.floor(_POS))
FRAC = np.float32(_POS - np.floor(_POS))

NC, NS = 2, 16
NW = NC * NS
PER_W = NTOT // NW
ROWS_PER_W = NROW // NW
RVECS = NCOL // 16
NB12 = 4096
NB3 = 256
INTMAX = np.int32(2**31 - 1)
SIGNBIT = np.int32(-2**31)


def _ikey16(x):
    u = lax.bitcast_convert_type(x, jnp.int32)
    s = lax.shift_right_logical(u, 31)
    return u ^ lax.shift_right_logical(0 - s, 1)


def _mesh():
    return plsc.VectorSubcoreMesh(core_axis_name="c", subcore_axis_name="s")


_SC_PARAMS = pltpu.CompilerParams(needs_layout_passes=False)


def _sc_pass1(flat):
    @functools.partial(
        pl.kernel, mesh=_mesh(), compiler_params=_SC_PARAMS,
        out_type=jax.ShapeDtypeStruct((NW, NB12 * 16), jnp.int32),
        scratch_types=[pltpu.VMEM((NCOL,), jnp.float32),
                       pltpu.VMEM((NB12 * 16,), jnp.int32)])
    def k(x_hbm, h_hbm, buf, hist):
        wid = lax.axis_index("s") * NC + lax.axis_index("c")
        lane = lax.iota(jnp.int32, 16)
        ones = jnp.ones((16,), jnp.int32)

        def zero(i):
            hist[pl.ds(i * 16, 16)] = jnp.zeros((16,), jnp.int32)
        plsc.parallel_loop(0, NB12, unroll=8)(zero)

        for rr in range(ROWS_PER_W):
            pltpu.sync_copy(x_hbm.at[wid * ROWS_PER_W + rr], buf)

            def body(i):
                ikey = _ikey16(buf[pl.ds(i * 16, 16)])
                b = lax.shift_right_arithmetic(ikey, 20) + 2048
                plsc.addupdate_scatter(hist, [lax.shift_left(b, 4) | lane],
                                       ones)
            plsc.parallel_loop(0, RVECS, unroll=8)(body)
        pltpu.sync_copy(hist, h_hbm.at[wid])

    return k(flat)


def _sc_pass2(flat, sel1):
    @functools.partial(
        pl.kernel, mesh=_mesh(), compiler_params=_SC_PARAMS,
        out_type=(jax.ShapeDtypeStruct((NW, NB12 * 16), jnp.int32),
                  jax.ShapeDtypeStruct((NW, 16), jnp.int32)),
        scratch_types=[pltpu.VMEM((NCOL,), jnp.float32),
                       pltpu.VMEM((NB12 * 16,), jnp.int32),
                       pltpu.VMEM((128,), jnp.int32),
                       pltpu.VMEM((16,), jnp.int32)])
    def k(x_hbm, sel_hbm, h_hbm, mn_hbm, buf, hist, selbuf, mnbuf):
        wid = lax.axis_index("s") * NC + lax.axis_index("c")
        lane = lax.iota(jnp.int32, 16)
        ones = jnp.ones((16,), jnp.int32)
        pltpu.sync_copy(sel_hbm.at[0], selbuf)
        b1t = selbuf[pl.ds(0, 16)]

        def zero(i):
            hist[pl.ds(i * 16, 16)] = jnp.zeros((16,), jnp.int32)
        plsc.parallel_loop(0, NB12, unroll=8)(zero)

        minv = jnp.full((16,), INTMAX, jnp.int32)
        for rr in range(ROWS_PER_W):
            pltpu.sync_copy(x_hbm.at[wid * ROWS_PER_W + rr], buf)

            def body(i, minv):
                ikey = _ikey16(buf[pl.ds(i * 16, 16)])
                b1 = lax.shift_right_arithmetic(ikey, 20) + 2048
                b2 = lax.shift_right_logical(ikey, 8) & 0xFFF
                plsc.addupdate_scatter(hist, [lax.shift_left(b2, 4) | lane],
                                       ones, mask=b1 == b1t)
                return jnp.minimum(minv, jnp.where(b1 > b1t, ikey, INTMAX))
            minv = plsc.parallel_loop(0, RVECS, unroll=8, carry=minv)(body)
        mnbuf[...] = minv
        pltpu.sync_copy(hist, h_hbm.at[wid])
        pltpu.sync_copy(mnbuf, mn_hbm.at[wid])

    return k(flat, sel1)


def _sc_pass3(flat, sel2):
    @functools.partial(
        pl.kernel, mesh=_mesh(), compiler_params=_SC_PARAMS,
        out_type=(jax.ShapeDtypeStruct((NW, NB3 * 16), jnp.int32),
                  jax.ShapeDtypeStruct((NW, 16), jnp.int32)),
        scratch_types=[pltpu.VMEM((NCOL,), jnp.float32),
                       pltpu.VMEM((NB3 * 16,), jnp.int32),
                       pltpu.VMEM((128,), jnp.int32),
                       pltpu.VMEM((128,), jnp.int32),
                       pltpu.VMEM((16,), jnp.int32)])
    def k(x_hbm, sel_hbm, h_hbm, mn_hbm, buf, hist, selb1, selb2, mnbuf):
        wid = lax.axis_index("s") * NC + lax.axis_index("c")
        lane = lax.iota(jnp.int32, 16)
        ones = jnp.ones((16,), jnp.int32)
        pltpu.sync_copy(sel_hbm.at[0], selb1)
        pltpu.sync_copy(sel_hbm.at[1], selb2)
        b1t = selb1[pl.ds(0, 16)]
        b2t = selb2[pl.ds(0, 16)]

        def zero(i):
            hist[pl.ds(i * 16, 16)] = jnp.zeros((16,), jnp.int32)
        plsc.parallel_loop(0, NB3, unroll=8)(zero)

        minv = jnp.full((16,), INTMAX, jnp.int32)
        for rr in range(ROWS_PER_W):
            pltpu.sync_copy(x_hbm.at[wid * ROWS_PER_W + rr], buf)

            def body(i, minv):
                ikey = _ikey16(buf[pl.ds(i * 16, 16)])
                b1 = lax.shift_right_arithmetic(ikey, 20) + 2048
                b2 = lax.shift_right_logical(ikey, 8) & 0xFFF
                b3 = ikey & 0xFF
                in1 = b1 == b1t
                plsc.addupdate_scatter(hist, [lax.shift_left(b3, 4) | lane],
                                       ones, mask=in1 & (b2 == b2t))
                return jnp.minimum(
                    minv, jnp.where(in1 & (b2 > b2t), ikey, INTMAX))
            minv = plsc.parallel_loop(0, RVECS, unroll=8, carry=minv)(body)
        mnbuf[...] = minv
        pltpu.sync_copy(hist, h_hbm.at[wid])
        pltpu.sync_copy(mnbuf, mn_hbm.at[wid])

    return k(flat, sel2)


def _select_math(h, R, kt):
    f32 = jnp.float32
    rows = lax.broadcasted_iota(jnp.int32, (R, 128), 0)
    cols = lax.broadcasted_iota(jnp.int32, (R, 128), 1)
    bucket = (rows * 8 + lax.shift_right_logical(cols, 4)).astype(f32)
    gi = lax.shift_right_logical(
        lax.broadcasted_iota(jnp.int32, (128, 128), 0), 4)
    gj = lax.shift_right_logical(
        lax.broadcasted_iota(jnp.int32, (128, 128), 1), 4)
    same = (gi == gj).astype(f32)
    before = (gi < gj).astype(f32)
    hb = jnp.dot(h, same, preferred_element_type=f32,
                 precision=lax.Precision.HIGHEST)
    win = jnp.dot(h, before, preferred_element_type=f32,
                  precision=lax.Precision.HIGHEST)
    ri = lax.broadcasted_iota(jnp.int32, (R, R), 0)
    rj = lax.broadcasted_iota(jnp.int32, (R, R), 1)
    lower = (ri > rj).astype(f32)
    rs = jnp.broadcast_to(jnp.sum(h, axis=1, keepdims=True), (R, 128))
    rex = jnp.dot(lower, rs, preferred_element_type=f32,
                  precision=lax.Precision.HIGHEST)
    cb = rex + win
    cond = (cb <= kt) & (kt < cb + hb)
    neg = jnp.float32(-1.0)
    return (jnp.max(jnp.where(cond, bucket, neg)),
            jnp.max(jnp.where(cond, cb, neg)),
            jnp.max(jnp.where(cond, hb, neg)))


def _rows_to_out(vals):
    r = lax.broadcasted_iota(jnp.int32, (8, 128), 0)
    out = jnp.zeros((8, 128), jnp.float32)
    for i, v in enumerate(vals):
        out = out + jnp.where(r == i, v, 0.0)
    return out.astype(jnp.int32)


def _tc_select1(h1v):
    def body(h_ref, o_ref):
        h = jnp.sum(h_ref[...].astype(jnp.float32), axis=0)
        b, rex, cnt = _select_math(h, NB12 // 8, jnp.float32(K_RANK))
        o_ref[...] = _rows_to_out([b, rex, cnt])

    return pl.pallas_call(
        body, out_shape=jax.ShapeDtypeStruct((8, 128), jnp.int32))(h1v)


def _tc_select2(h2v, sel1):
    def body(h_ref, s_ref, o_ref):
        h = jnp.sum(h_ref[...].astype(jnp.float32), axis=0)
        b1 = s_ref[0, 0]
        r0 = s_ref[1, 0]
        kt = (K_RANK - r0).astype(jnp.float32)
        b2, rex, cnt = _select_math(h, NB12 // 8, kt)
        r01 = r0.astype(jnp.float32) + rex
        o_ref[...] = _rows_to_out([b1.astype(jnp.float32), b2, r01, cnt])

    return pl.pallas_call(
        body, out_shape=jax.ShapeDtypeStruct((8, 128), jnp.int32))(h2v, sel1)


def _tofloat(ik):
    bits = jnp.where(ik >= 0, ik, (~ik) | SIGNBIT)
    return lax.bitcast_convert_type(bits, jnp.float32)


def _tc_finalize(tensor, h3v, sel2, mina):
    grid = 16
    rows_blk = NROW // grid

    def body(x_ref, h_ref, s_ref, m_ref, o_ref):
        h = jnp.sum(h_ref[...].astype(jnp.float32), axis=0)
        b1 = s_ref[0, 0]
        b2 = s_ref[1, 0]
        r01 = s_ref[2, 0]
        cnt12 = s_ref[3, 0]
        jt = (K_RANK - r01).astype(jnp.float32)
        b3a, _, _ = _select_math(h, NB3 // 8, jt)
        b3b, _, _ = _select_math(h, NB3 // 8, jt + 1.0)
        prefix = (b1 - 2048) * 1048576 + b2 * 256
        ikey_k = prefix + b3a.astype(jnp.int32)
        ikey_k1_in = prefix + b3b.astype(jnp.int32)
        mmin = jnp.min(m_ref[...])
        have_b = (jt + 1.0) < cnt12.astype(jnp.float32)
        ikey_k1 = jnp.where(have_b, ikey_k1_in, mmin)
        vk = _tofloat(ikey_k)
        vk1 = _tofloat(ikey_k1)
        t = vk * (np.float32(1.0) - FRAC) + vk1 * FRAC
        clip = jnp.where(vk1 <= t, vk1, vk)
        o_ref[...] = jnp.minimum(x_ref[...], clip)

    return pl.pallas_call(
        body,
        grid=(grid,),
        in_specs=[
            pl.BlockSpec((rows_blk, NCOL), lambda i: (i, 0)),
            pl.BlockSpec((NW, NB3 * 16 // 128, 128), lambda i: (0, 0, 0)),
            pl.BlockSpec((8, 128), lambda i: (0, 0)),
            pl.BlockSpec((8, 128), lambda i: (0, 0)),
        ],
        out_specs=pl.BlockSpec((rows_blk, NCOL), lambda i: (i, 0)),
        out_shape=jax.ShapeDtypeStruct((NROW, NCOL), jnp.float32),
    )(tensor, h3v, sel2, mina)


def kernel(tensor):
    h1 = _sc_pass1(tensor)
    sel1 = _tc_select1(h1.reshape(NW, NB12 * 16 // 128, 128))
    h2, mina2 = _sc_pass2(tensor, sel1)
    sel2 = _tc_select2(h2.reshape(NW, NB12 * 16 // 128, 128), sel1)
    h3, mina3 = _sc_pass3(tensor, sel2)
    mina = jnp.concatenate([mina2, mina3], axis=0).reshape(8, 128)
    return _tc_finalize(tensor, h3.reshape(NW, NB3 * 16 // 128, 128),
                        sel2, mina)


# speedup vs baseline: 41.7111x; 1.1822x over previous
import functools

import numpy as np
import jax
import jax.numpy as jnp
from jax import lax
from jax.experimental import pallas as pl
from jax.experimental.pallas import tpu as pltpu
from jax.experimental.pallas import tpu_sc as plsc

NROW, NCOL = 128, 32768
NTOT = NROW * NCOL
_POS = np.float32(0.96) * np.float32(NTOT - 1)
K_RANK = int(np.floor(_POS))
FRAC = np.float32(_POS - np.floor(_POS))

NC, NS = 2, 16
NW = NC * NS
ROWS_PER_W = NROW // NW
CH = 16384
CVECS = CH // 16
NB12 = 4096
NB3 = 256
HR12 = NB12 * 16 // 128
HR3 = NB3 * 16 // 128
INTMAX = np.int32(2**31 - 1)
SIGNBIT = np.int32(-2**31)

_SC_PARAMS = pltpu.CompilerParams(needs_layout_passes=False)


def _mesh():
    return plsc.VectorSubcoreMesh(core_axis_name="c", subcore_axis_name="s")


def _keyu16(x):
    u = lax.bitcast_convert_type(x, jnp.int32)
    m = lax.shift_right_arithmetic(u, 31) | SIGNBIT
    return u ^ m


def _chunk_plan(wid):
    return [(wid * ROWS_PER_W + rr, off)
            for rr in range(ROWS_PER_W)
            for off in range(0, NCOL, CH)]


def _pipelined_rows(x_hbm, wid, bufs, sems, process):
    plan = _chunk_plan(wid)
    desc = [None, None]
    desc[0] = pltpu.async_copy(
        x_hbm.at[plan[0][0], pl.ds(plan[0][1], CH)], bufs[0], sems[0])
    for ci in range(len(plan)):
        nxt = ci + 1
        if nxt < len(plan):
            desc[nxt % 2] = pltpu.async_copy(
                x_hbm.at[plan[nxt][0], pl.ds(plan[nxt][1], CH)],
                bufs[nxt % 2], sems[nxt % 2])
        desc[ci % 2].wait()
        process(bufs[ci % 2])


def _sc_pass1(tensor, zeros12):
    @functools.partial(
        pl.kernel, mesh=_mesh(), compiler_params=_SC_PARAMS,
        out_type=jax.ShapeDtypeStruct((NW, HR12, 128), jnp.int32),
        scratch_types=[pltpu.VMEM((CH,), jnp.float32),
                       pltpu.VMEM((CH,), jnp.float32),
                       pltpu.VMEM((HR12, 128), jnp.int32),
                       pltpu.SemaphoreType.DMA,
                       pltpu.SemaphoreType.DMA])
    def k(x_hbm, z_hbm, h_hbm, buf0, buf1, hist, sem0, sem1):
        wid = lax.axis_index("s") * NC + lax.axis_index("c")
        lane = lax.iota(jnp.int32, 16)
        ones = jnp.ones((16,), jnp.int32)
        pltpu.sync_copy(z_hbm, hist)

        def process(buf):
            def body(i):
                keyu = _keyu16(buf[pl.ds(i * 16, 16)])
                b = lax.shift_right_logical(keyu, 20)
                row = lax.shift_right_logical(b, 3)
                col = lax.shift_left(b & 7, 4) | lane
                plsc.addupdate_scatter(hist, [row, col], ones)
            plsc.parallel_loop(0, CVECS, unroll=8)(body)

        _pipelined_rows(x_hbm, wid, (buf0, buf1), (sem0, sem1), process)
        pltpu.sync_copy(hist, h_hbm.at[wid])

    return k(tensor, zeros12)


def _sc_pass2(tensor, zeros12, sel1):
    @functools.partial(
        pl.kernel, mesh=_mesh(), compiler_params=_SC_PARAMS,
        out_type=(jax.ShapeDtypeStruct((NW, HR12, 128), jnp.int32),
                  jax.ShapeDtypeStruct((NW, 16), jnp.int32)),
        scratch_types=[pltpu.VMEM((CH,), jnp.float32),
                       pltpu.VMEM((CH,), jnp.float32),
                       pltpu.VMEM((HR12, 128), jnp.int32),
                       pltpu.VMEM((128,), jnp.int32),
                       pltpu.VMEM((16,), jnp.int32),
                       pltpu.SemaphoreType.DMA,
                       pltpu.SemaphoreType.DMA])
    def k(x_hbm, z_hbm, sel_hbm, h_hbm, mn_hbm, buf0, buf1, hist, selbuf,
          mnbuf, sem0, sem1):
        wid = lax.axis_index("s") * NC + lax.axis_index("c")
        lane = lax.iota(jnp.int32, 16)
        ones = jnp.ones((16,), jnp.int32)
        pltpu.sync_copy(sel_hbm.at[0], selbuf)
        b1t = selbuf[pl.ds(0, 16)]
        pltpu.sync_copy(z_hbm, hist)

        minv_box = [jnp.full((16,), INTMAX, jnp.int32)]

        def process(buf):
            def body(i, minv):
                keyu = _keyu16(buf[pl.ds(i * 16, 16)])
                b1 = lax.shift_right_logical(keyu, 20)
                b2 = lax.shift_right_logical(keyu, 8) & 0xFFF
                row = lax.shift_right_logical(b2, 3)
                col = lax.shift_left(b2 & 7, 4) | lane
                plsc.addupdate_scatter(hist, [row, col], ones,
                                       mask=b1 == b1t)
                ikey = keyu ^ SIGNBIT
                return jnp.minimum(minv,
                                   jnp.where(b1 > b1t, ikey, INTMAX))
            minv_box[0] = plsc.parallel_loop(
                0, CVECS, unroll=8, carry=minv_box[0])(body)

        _pipelined_rows(x_hbm, wid, (buf0, buf1), (sem0, sem1), process)
        mnbuf[...] = minv_box[0]
        pltpu.sync_copy(hist, h_hbm.at[wid])
        pltpu.sync_copy(mnbuf, mn_hbm.at[wid])

    return k(tensor, zeros12, sel1)


def _sc_pass3(tensor, zeros3, sel2):
    @functools.partial(
        pl.kernel, mesh=_mesh(), compiler_params=_SC_PARAMS,
        out_type=(jax.ShapeDtypeStruct((NW, HR3, 128), jnp.int32),
                  jax.ShapeDtypeStruct((NW, 16), jnp.int32)),
        scratch_types=[pltpu.VMEM((CH,), jnp.float32),
                       pltpu.VMEM((CH,), jnp.float32),
                       pltpu.VMEM((HR3, 128), jnp.int32),
                       pltpu.VMEM((128,), jnp.int32),
                       pltpu.VMEM((128,), jnp.int32),
                       pltpu.VMEM((16,), jnp.int32),
                       pltpu.SemaphoreType.DMA,
                       pltpu.SemaphoreType.DMA])
    def k(x_hbm, z_hbm, sel_hbm, h_hbm, mn_hbm, buf0, buf1, hist, selb1,
          selb2, mnbuf, sem0, sem1):
        wid = lax.axis_index("s") * NC + lax.axis_index("c")
        lane = lax.iota(jnp.int32, 16)
        ones = jnp.ones((16,), jnp.int32)
        pltpu.sync_copy(sel_hbm.at[0], selb1)
        pltpu.sync_copy(sel_hbm.at[1], selb2)
        b1t = selb1[pl.ds(0, 16)]
        b2t = selb2[pl.ds(0, 16)]
        pltpu.sync_copy(z_hbm, hist)

        minv_box = [jnp.full((16,), INTMAX, jnp.int32)]

        def process(buf):
            def body(i, minv):
                keyu = _keyu16(buf[pl.ds(i * 16, 16)])
                b1 = lax.shift_right_logical(keyu, 20)
                b2 = lax.shift_right_logical(keyu, 8) & 0xFFF
                b3 = keyu & 0xFF
                row = lax.shift_right_logical(b3, 3)
                col = lax.shift_left(b3 & 7, 4) | lane
                in1 = b1 == b1t
                plsc.addupdate_scatter(hist, [row, col], ones,
                                       mask=in1 & (b2 == b2t))
                ikey = keyu ^ SIGNBIT
                return jnp.minimum(
                    minv, jnp.where(in1 & (b2 > b2t), ikey, INTMAX))
            minv_box[0] = plsc.parallel_loop(
                0, CVECS, unroll=8, carry=minv_box[0])(body)

        _pipelined_rows(x_hbm, wid, (buf0, buf1), (sem0, sem1), process)
        mnbuf[...] = minv_box[0]
        pltpu.sync_copy(hist, h_hbm.at[wid])
        pltpu.sync_copy(mnbuf, mn_hbm.at[wid])

    return k(tensor, zeros3, sel2)


def _select_math(h, R, kt):
    f32 = jnp.float32
    hp = lax.Precision.HIGHEST
    rows = lax.broadcasted_iota(jnp.int32, (R, 128), 0)
    cols = lax.broadcasted_iota(jnp.int32, (R, 128), 1)
    bucket = (rows * 8 + lax.shift_right_logical(cols, 4)).astype(f32)
    gi = lax.shift_right_logical(
        lax.broadcasted_iota(jnp.int32, (128, 128), 0), 4)
    gj = lax.shift_right_logical(
        lax.broadcasted_iota(jnp.int32, (128, 128), 1), 4)
    same = (gi == gj).astype(f32)
    before = (gi < gj).astype(f32)
    hb = jnp.dot(h, same, preferred_element_type=f32, precision=hp)
    win = jnp.dot(h, before, preferred_element_type=f32, precision=hp)
    ri = lax.broadcasted_iota(jnp.int32, (R, R), 0)
    rj = lax.broadcasted_iota(jnp.int32, (R, R), 1)
    lower = (ri > rj).astype(f32)
    rs = jnp.broadcast_to(jnp.sum(h, axis=1, keepdims=True), (R, 128))
    rex = jnp.dot(lower, rs, preferred_element_type=f32, precision=hp)
    cb = rex + win
    cond = (cb <= kt) & (kt < cb + hb)
    neg = jnp.float32(-1.0)
    return (jnp.max(jnp.where(cond, bucket, neg)),
            jnp.max(jnp.where(cond, cb, neg)),
            jnp.max(jnp.where(cond, hb, neg)))


def _rows_to_out(vals):
    r = lax.broadcasted_iota(jnp.int32, (8, 128), 0)
    out = jnp.zeros((8, 128), jnp.float32)
    for i, v in enumerate(vals):
        out = out + jnp.where(r == i, v, 0.0)
    return out.astype(jnp.int32)


def _tc_select1(h1v):
    def body(h_ref, o_ref):
        h = jnp.sum(h_ref[...].astype(jnp.float32), axis=0)
        b, rex, cnt = _select_math(h, HR12, jnp.float32(K_RANK))
        o_ref[...] = _rows_to_out([b, rex, cnt])

    return pl.pallas_call(
        body, out_shape=jax.ShapeDtypeStruct((8, 128), jnp.int32))(h1v)


def _tc_select2(h2v, sel1):
    def body(h_ref, s_ref, o_ref):
        h = jnp.sum(h_ref[...].astype(jnp.float32), axis=0)
        b1 = s_ref[0, 0]
        r0 = s_ref[1, 0]
        kt = (K_RANK - r0).astype(jnp.float32)
        b2, rex, cnt = _select_math(h, HR12, kt)
        r01 = r0.astype(jnp.float32) + rex
        o_ref[...] = _rows_to_out([b1.astype(jnp.float32), b2, r01, cnt])

    return pl.pallas_call(
        body, out_shape=jax.ShapeDtypeStruct((8, 128), jnp.int32))(h2v, sel1)


def _tofloat(ik):
    bits = jnp.where(ik >= 0, ik, (~ik) | SIGNBIT)
    return lax.bitcast_convert_type(bits, jnp.float32)


def _tc_finalize(tensor, h3v, sel2, mina):
    grid = 16
    rows_blk = NROW // grid

    def body(x_ref, h_ref, s_ref, m_ref, o_ref):
        h = jnp.sum(h_ref[...].astype(jnp.float32), axis=0)
        b1 = s_ref[0, 0]
        b2 = s_ref[1, 0]
        r01 = s_ref[2, 0]
        cnt12 = s_ref[3, 0]
        jt = (K_RANK - r01).astype(jnp.float32)
        b3a, _, _ = _select_math(h, HR3, jt)
        b3b, _, _ = _select_math(h, HR3, jt + 1.0)
        prefix = (b1 - 2048) * 1048576 + b2 * 256
        ikey_k = prefix + b3a.astype(jnp.int32)
        ikey_k1_in = prefix + b3b.astype(jnp.int32)
        mmin = jnp.min(m_ref[...])
        have_b = (jt + 1.0) < cnt12.astype(jnp.float32)
        ikey_k1 = jnp.where(have_b, ikey_k1_in, mmin)
        vk = _tofloat(ikey_k)
        vk1 = _tofloat(ikey_k1)
        t = vk * (np.float32(1.0) - FRAC) + vk1 * FRAC
        clip = jnp.where(vk1 <= t, vk1, vk)
        o_ref[...] = jnp.minimum(x_ref[...], clip)

    return pl.pallas_call(
        body,
        grid=(grid,),
        in_specs=[
            pl.BlockSpec((rows_blk, NCOL), lambda i: (i, 0)),
            pl.BlockSpec((NW, HR3, 128), lambda i: (0, 0, 0)),
            pl.BlockSpec((8, 128), lambda i: (0, 0)),
            pl.BlockSpec((8, 128), lambda i: (0, 0)),
        ],
        out_specs=pl.BlockSpec((rows_blk, NCOL), lambda i: (i, 0)),
        out_shape=jax.ShapeDtypeStruct((NROW, NCOL), jnp.float32),
    )(tensor, h3v, sel2, mina)


def kernel(tensor):
    zeros12 = jnp.zeros((HR12, 128), jnp.int32)
    zeros3 = jnp.zeros((HR3, 128), jnp.int32)
    h1 = _sc_pass1(tensor, zeros12)
    sel1 = _tc_select1(h1)
    h2, mina2 = _sc_pass2(tensor, zeros12, sel1)
    sel2 = _tc_select2(h2, sel1)
    h3, mina3 = _sc_pass3(tensor, zeros3, sel2)
    mina = jnp.concatenate([mina2, mina3], axis=0).reshape(8, 128)
    return _tc_finalize(tensor, h3, sel2, mina)


# speedup vs baseline: 46.7480x; 1.1208x over previous
import functools

import numpy as np
import jax
import jax.numpy as jnp
from jax import lax
from jax.experimental import pallas as pl
from jax.experimental.pallas import tpu as pltpu
from jax.experimental.pallas import tpu_sc as plsc

NROW, NCOL = 128, 32768
NTOT = NROW * NCOL
_POS = np.float32(0.96) * np.float32(NTOT - 1)
K_RANK = int(np.floor(_POS))
FRAC = np.float32(_POS - np.floor(_POS))

NC, NS = 2, 16
NW = NC * NS
ROWS_PER_W = NROW // NW
CH = 16384
CVECS = CH // 16
NB12 = 4096
NB3 = 256
HR12 = NB12 // 128
HR3 = NB3 // 128
INTMAX = np.int32(2**31 - 1)
SIGNBIT = np.int32(-2**31)

_SC_PARAMS = pltpu.CompilerParams(needs_layout_passes=False)


def _mesh():
    return plsc.VectorSubcoreMesh(core_axis_name="c", subcore_axis_name="s")


def _keyu16(x):
    u = lax.bitcast_convert_type(x, jnp.int32)
    m = lax.shift_right_arithmetic(u, 31) | SIGNBIT
    return u ^ m


def _chunk_plan(wid):
    return [(wid * ROWS_PER_W + rr, off)
            for rr in range(ROWS_PER_W)
            for off in range(0, NCOL, CH)]


def _pipelined_rows(x_hbm, wid, bufs, sems, process):
    plan = _chunk_plan(wid)
    desc = [None, None]
    desc[0] = pltpu.async_copy(
        x_hbm.at[plan[0][0], pl.ds(plan[0][1], CH)], bufs[0], sems[0])
    for ci in range(len(plan)):
        nxt = ci + 1
        if nxt < len(plan):
            desc[nxt % 2] = pltpu.async_copy(
                x_hbm.at[plan[nxt][0], pl.ds(plan[nxt][1], CH)],
                bufs[nxt % 2], sems[nxt % 2])
        desc[ci % 2].wait()
        process(bufs[ci % 2])


def _sc_pass1(tensor, zeros12):
    @functools.partial(
        pl.kernel, mesh=_mesh(), compiler_params=_SC_PARAMS,
        out_type=jax.ShapeDtypeStruct((NW * 16, HR12, 128), jnp.int32),
        scratch_types=[pltpu.VMEM((CH,), jnp.float32),
                       pltpu.VMEM((CH,), jnp.float32),
                       pltpu.VMEM((16, HR12, 128), jnp.int32),
                       pltpu.SemaphoreType.DMA,
                       pltpu.SemaphoreType.DMA])
    def k(x_hbm, z_hbm, h_hbm, buf0, buf1, hist, sem0, sem1):
        wid = lax.axis_index("s") * NC + lax.axis_index("c")
        lane = lax.iota(jnp.int32, 16)
        ones = jnp.ones((16,), jnp.int32)
        pltpu.sync_copy(z_hbm, hist)

        def process(buf):
            def body(i):
                keyu = _keyu16(buf[pl.ds(i * 16, 16)])
                b = lax.shift_right_logical(keyu, 20)
                row = lax.shift_right_logical(b, 7)
                col = b & 127
                plsc.addupdate_scatter(hist, [lane, row, col], ones)
            plsc.parallel_loop(0, CVECS, unroll=8)(body)

        _pipelined_rows(x_hbm, wid, (buf0, buf1), (sem0, sem1), process)
        pltpu.sync_copy(hist, h_hbm.at[pl.ds(wid * 16, 16)])

    return k(tensor, zeros12)


def _sc_pass2(tensor, zeros12, sel1):
    @functools.partial(
        pl.kernel, mesh=_mesh(), compiler_params=_SC_PARAMS,
        out_type=jax.ShapeDtypeStruct((NW * 16, HR12, 128), jnp.int32),
        scratch_types=[pltpu.VMEM((CH,), jnp.float32),
                       pltpu.VMEM((CH,), jnp.float32),
                       pltpu.VMEM((16, HR12, 128), jnp.int32),
                       pltpu.VMEM((128,), jnp.int32),
                       pltpu.SemaphoreType.DMA,
                       pltpu.SemaphoreType.DMA])
    def k(x_hbm, z_hbm, sel_hbm, h_hbm, buf0, buf1, hist, selbuf,
          sem0, sem1):
        wid = lax.axis_index("s") * NC + lax.axis_index("c")
        lane = lax.iota(jnp.int32, 16)
        ones = jnp.ones((16,), jnp.int32)
        pltpu.sync_copy(sel_hbm.at[0], selbuf)
        b1t = selbuf[pl.ds(0, 16)]
        pltpu.sync_copy(z_hbm, hist)

        def process(buf):
            def body(i):
                keyu = _keyu16(buf[pl.ds(i * 16, 16)])
                b1 = lax.shift_right_logical(keyu, 20)
                row = lax.shift_right_logical(keyu, 15) & 31
                col = lax.shift_right_logical(keyu, 8) & 127
                plsc.addupdate_scatter(hist, [lane, row, col], ones,
                                       mask=b1 == b1t)
            plsc.parallel_loop(0, CVECS, unroll=8)(body)

        _pipelined_rows(x_hbm, wid, (buf0, buf1), (sem0, sem1), process)
        pltpu.sync_copy(hist, h_hbm.at[pl.ds(wid * 16, 16)])

    return k(tensor, zeros12, sel1)


def _sc_pass3(tensor, zeros3, sel2):
    @functools.partial(
        pl.kernel, mesh=_mesh(), compiler_params=_SC_PARAMS,
        out_type=(jax.ShapeDtypeStruct((NW * 16, HR3, 128), jnp.int32),
                  jax.ShapeDtypeStruct((NW, 16), jnp.int32)),
        scratch_types=[pltpu.VMEM((CH,), jnp.float32),
                       pltpu.VMEM((CH,), jnp.float32),
                       pltpu.VMEM((16, HR3, 128), jnp.int32),
                       pltpu.VMEM((128,), jnp.int32),
                       pltpu.VMEM((128,), jnp.int32),
                       pltpu.VMEM((16,), jnp.int32),
                       pltpu.SemaphoreType.DMA,
                       pltpu.SemaphoreType.DMA])
    def k(x_hbm, z_hbm, sel_hbm, h_hbm, mn_hbm, buf0, buf1, hist, selb1,
          selb2, mnbuf, sem0, sem1):
        wid = lax.axis_index("s") * NC + lax.axis_index("c")
        lane = lax.iota(jnp.int32, 16)
        ones = jnp.ones((16,), jnp.int32)
        pltpu.sync_copy(sel_hbm.at[0], selb1)
        pltpu.sync_copy(sel_hbm.at[1], selb2)
        b1t = selb1[pl.ds(0, 16)]
        b2t = selb2[pl.ds(0, 16)]
        hi24t = lax.shift_left(b1t, 12) | b2t
        t1s = (lax.shift_left(hi24t, 8) | 255) ^ SIGNBIT
        pltpu.sync_copy(z_hbm, hist)

        minv_box = [jnp.full((16,), INTMAX, jnp.int32)]

        def process(buf):
            def body(i, minv):
                keyu = _keyu16(buf[pl.ds(i * 16, 16)])
                hi24 = lax.shift_right_logical(keyu, 8)
                row = lax.shift_right_logical(keyu, 7) & 1
                col = keyu & 127
                plsc.addupdate_scatter(hist, [lane, row, col], ones,
                                       mask=hi24 == hi24t)
                ikey = keyu ^ SIGNBIT
                return jnp.minimum(
                    minv, jnp.where(ikey > t1s, ikey, INTMAX))
            minv_box[0] = plsc.parallel_loop(
                0, CVECS, unroll=8, carry=minv_box[0])(body)

        _pipelined_rows(x_hbm, wid, (buf0, buf1), (sem0, sem1), process)
        mnbuf[...] = minv_box[0]
        pltpu.sync_copy(hist, h_hbm.at[pl.ds(wid * 16, 16)])
        pltpu.sync_copy(mnbuf, mn_hbm.at[wid])

    return k(tensor, zeros3, sel2)


def _select_math(h, R, kt):
    f32 = jnp.float32
    hp = lax.Precision.HIGHEST
    rows = lax.broadcasted_iota(jnp.int32, (R, 128), 0)
    cols = lax.broadcasted_iota(jnp.int32, (R, 128), 1)
    bucket = (rows * 128 + cols).astype(f32)
    ci = lax.broadcasted_iota(jnp.int32, (128, 128), 0)
    cj = lax.broadcasted_iota(jnp.int32, (128, 128), 1)
    before = (ci < cj).astype(f32)
    win = jnp.dot(h, before, preferred_element_type=f32, precision=hp)
    ri = lax.broadcasted_iota(jnp.int32, (R, R), 0)
    rj = lax.broadcasted_iota(jnp.int32, (R, R), 1)
    lower = (ri > rj).astype(f32)
    rs = jnp.broadcast_to(jnp.sum(h, axis=1, keepdims=True), (R, 128))
    rex = jnp.dot(lower, rs, preferred_element_type=f32, precision=hp)
    cb = rex + win
    cond = (cb <= kt) & (kt < cb + h)
    neg = jnp.float32(-1.0)
    return (jnp.max(jnp.where(cond, bucket, neg)),
            jnp.max(jnp.where(cond, cb, neg)),
            jnp.max(jnp.where(cond, h, neg)))


def _rows_to_out(vals):
    r = lax.broadcasted_iota(jnp.int32, (8, 128), 0)
    out = jnp.zeros((8, 128), jnp.float32)
    for i, v in enumerate(vals):
        out = out + jnp.where(r == i, v, 0.0)
    return out.astype(jnp.int32)


def _tc_select1(h1v):
    def body(h_ref, o_ref):
        h = jnp.sum(h_ref[...].astype(jnp.float32), axis=0)
        b, rex, cnt = _select_math(h, HR12, jnp.float32(K_RANK))
        o_ref[...] = _rows_to_out([b, rex, cnt])

    return pl.pallas_call(
        body, out_shape=jax.ShapeDtypeStruct((8, 128), jnp.int32))(h1v)


def _tc_select2(h2v, sel1):
    def body(h_ref, s_ref, o_ref):
        h = jnp.sum(h_ref[...].astype(jnp.float32), axis=0)
        b1 = s_ref[0, 0]
        r0 = s_ref[1, 0]
        kt = (K_RANK - r0).astype(jnp.float32)
        b2, rex, cnt = _select_math(h, HR12, kt)
        r01 = r0.astype(jnp.float32) + rex
        o_ref[...] = _rows_to_out([b1.astype(jnp.float32), b2, r01, cnt])

    return pl.pallas_call(
        body, out_shape=jax.ShapeDtypeStruct((8, 128), jnp.int32))(h2v, sel1)


def _tofloat(ik):
    bits = jnp.where(ik >= 0, ik, (~ik) | SIGNBIT)
    return lax.bitcast_convert_type(bits, jnp.float32)


def _tc_finalize(tensor, h3v, sel2, mina):
    grid = 16
    rows_blk = NROW // grid

    def body(x_ref, h_ref, s_ref, m_ref, o_ref):
        h = jnp.sum(h_ref[...].astype(jnp.float32), axis=0)
        b1 = s_ref[0, 0]
        b2 = s_ref[1, 0]
        r01 = s_ref[2, 0]
        cnt12 = s_ref[3, 0]
        jt = (K_RANK - r01).astype(jnp.float32)
        b3a, _, _ = _select_math(h, HR3, jt)
        b3b, _, _ = _select_math(h, HR3, jt + 1.0)
        prefix = (b1 - 2048) * 1048576 + b2 * 256
        ikey_k = prefix + b3a.astype(jnp.int32)
        ikey_k1_in = prefix + b3b.astype(jnp.int32)
        mmin = jnp.min(m_ref[...])
        have_b = (jt + 1.0) < cnt12.astype(jnp.float32)
        ikey_k1 = jnp.where(have_b, ikey_k1_in, mmin)
        vk = _tofloat(ikey_k)
        vk1 = _tofloat(ikey_k1)
        t = vk * (np.float32(1.0) - FRAC) + vk1 * FRAC
        clip = jnp.where(vk1 <= t, vk1, vk)
        o_ref[...] = jnp.minimum(x_ref[...], clip)

    return pl.pallas_call(
        body,
        grid=(grid,),
        in_specs=[
            pl.BlockSpec((rows_blk, NCOL), lambda i: (i, 0)),
            pl.BlockSpec((NW * 16, HR3, 128), lambda i: (0, 0, 0)),
            pl.BlockSpec((8, 128), lambda i: (0, 0)),
            pl.BlockSpec((4, 128), lambda i: (0, 0)),
        ],
        out_specs=pl.BlockSpec((rows_blk, NCOL), lambda i: (i, 0)),
        out_shape=jax.ShapeDtypeStruct((NROW, NCOL), jnp.float32),
    )(tensor, h3v, sel2, mina)


def kernel(tensor):
    zeros12 = jnp.zeros((16, HR12, 128), jnp.int32)
    zeros3 = jnp.zeros((16, HR3, 128), jnp.int32)
    h1 = _sc_pass1(tensor, zeros12)
    sel1 = _tc_select1(h1)
    h2 = _sc_pass2(tensor, zeros12, sel1)
    sel2 = _tc_select2(h2, sel1)
    h3, mina3 = _sc_pass3(tensor, zeros3, sel2)
    mina = mina3.reshape(4, 128)
    return _tc_finalize(tensor, h3, sel2, mina)
